# baseline (speedup 1.0000x reference)
import functools

import jax
import jax.numpy as jnp
from jax import lax
from jax.experimental import pallas as pl
from jax.experimental.pallas import tpu as pltpu
from jax.experimental.pallas import tpu_sc as plsc

F32 = jnp.float32
I32 = jnp.int32
NC, NS, NLANE = 2, 16, 16
NW = NC * NS
CHUNK = 128


def _align(n, m):
    return ((n + m - 1) // m) * m


def _embed(x, W, A, blk=1024):
    Np, K = x.shape

    def body(x_ref, w_ref, a_ref, h_ref, s_ref):
        h = jnp.dot(x_ref[...], w_ref[...], preferred_element_type=F32)
        h_ref[...] = h
        s_ref[...] = jnp.dot(h, a_ref[...], preferred_element_type=F32)

    return pl.pallas_call(
        body,
        grid=(Np // blk,),
        in_specs=[pl.BlockSpec((blk, K), lambda i: (i, 0)),
                  pl.BlockSpec((K, 64), lambda i: (0, 0)),
                  pl.BlockSpec((64, 16), lambda i: (0, 0))],
        out_specs=[pl.BlockSpec((blk, 64), lambda i: (i, 0)),
                   pl.BlockSpec((blk, 16), lambda i: (i, 0))],
        out_shape=[jax.ShapeDtypeStruct((Np, 64), F32),
                   jax.ShapeDtypeStruct((Np, 16), F32)],
    )(x, W, A)


def _finish(numer4, denom, tvec, n_real, blk=1024):
    _, nslab, Np, SW = numer4.shape
    grid = Np // blk

    def body(*refs):
        nrefs = refs[:nslab]
        d_ref, tv_ref, y_ref, t_ref, st_ref = refs[nslab:]
        i = pl.program_id(0)
        n = jnp.concatenate([r[0, 0] + r[1, 0] for r in nrefs], axis=-1)
        d = d_ref[0] + d_ref[1] + 1e-16
        y = n / d
        y_ref[...] = y
        t_ref[...] = jnp.dot(y, tv_ref[...], preferred_element_type=F32)
        rows = i * blk + lax.broadcasted_iota(I32, (blk, 64), 0)
        ym = jnp.where(rows < n_real, y, 0.0)

        @pl.when(i == 0)
        def _():
            st_ref[...] = jnp.zeros((8, 128), F32)

        st_ref[0:1, 0:64] += jnp.sum(ym, axis=0, keepdims=True)
        st_ref[1:2, 0:64] += jnp.sum(ym * ym, axis=0, keepdims=True)

    in_specs = [pl.BlockSpec((NC, 1, blk, SW), (lambda i, p=p: (0, p, i, 0)))
                for p in range(nslab)]
    in_specs += [pl.BlockSpec((2, blk, 1), lambda i: (0, i, 0)),
                 pl.BlockSpec((64, 1), lambda i: (0, 0))]
    return pl.pallas_call(
        body,
        grid=(grid,),
        in_specs=in_specs,
        out_specs=[pl.BlockSpec((blk, 64), lambda i: (i, 0)),
                   pl.BlockSpec((blk, 1), lambda i: (i, 0)),
                   pl.BlockSpec((8, 128), lambda i: (0, 0))],
        out_shape=[jax.ShapeDtypeStruct((Np, 64), F32),
                   jax.ShapeDtypeStruct((Np, 1), F32),
                   jax.ShapeDtypeStruct((8, 128), F32)],
    )(*([numer4] * nslab), denom.reshape(2, Np, 1), tvec.reshape(64, 1))


def _bn_relu(y, stats, g, b, n_real, W2=None, A2=None, blk=1024):
    Np, _ = y.shape
    project = W2 is not None
    inv_n = 1.0 / float(n_real)

    def body(y_ref, st_ref, g_ref, b_ref, *rest):
        if project:
            w_ref, a_ref, z_ref, za_ref = rest
        else:
            (z_ref,) = rest
        s = st_ref[0:1, 0:64]
        sq = st_ref[1:2, 0:64]
        m = s * inv_n
        var = sq * inv_n - m * m
        r = lax.rsqrt(var + 1e-5)
        p = jnp.maximum((y_ref[...] - m) * (r * g_ref[...]) + b_ref[...], 0.0)
        if project:
            z = jnp.dot(p, w_ref[...], preferred_element_type=F32)
            z_ref[...] = z
            za_ref[...] = jnp.dot(z, a_ref[...], preferred_element_type=F32)
        else:
            z_ref[...] = p

    in_specs = [pl.BlockSpec((blk, 64), lambda i: (i, 0)),
                pl.BlockSpec((8, 128), lambda i: (0, 0)),
                pl.BlockSpec((1, 64), lambda i: (0, 0)),
                pl.BlockSpec((1, 64), lambda i: (0, 0))]
    args = [y, stats, g.reshape(1, 64), b.reshape(1, 64)]
    out_specs = [pl.BlockSpec((blk, 64), lambda i: (i, 0))]
    out_shape = [jax.ShapeDtypeStruct((Np, 64), F32)]
    if project:
        in_specs += [pl.BlockSpec((64, 64), lambda i: (0, 0)),
                     pl.BlockSpec((64, 16), lambda i: (0, 0))]
        args += [W2, A2]
        out_specs += [pl.BlockSpec((blk, 16), lambda i: (i, 0))]
        out_shape += [jax.ShapeDtypeStruct((Np, 16), F32)]

    res = pl.pallas_call(
        body, grid=(Np // blk,), in_specs=in_specs, out_specs=out_specs,
        out_shape=out_shape)(*args)
    return res if project else res[0]


def _edge_weights(nch, Na, Nb, Npad, two):
    mesh = plsc.VectorSubcoreMesh(core_axis_name="c", subcore_axis_name="s")
    stripe = Npad // NS
    ta_str = Na // NS
    tb_str = (Nb // NS) if two else 0

    tsz = max(stripe, ta_str, tb_str)
    scratch = [pltpu.VMEM((nch, CHUNK), I32),
               pltpu.VMEM((nch, CHUNK), I32),
               pltpu.VMEM((CHUNK,), F32),
               pltpu.VMEM((CHUNK,), F32),
               pltpu.VMEM((CHUNK,), F32),
               pltpu.VMEM((tsz,), F32),
               pltpu.VMEM_SHARED((Na,), F32),
               pltpu.VMEM_SHARED((Npad,), F32),
               pltpu.SemaphoreType.DMA]
    if two:
        scratch.insert(1, pltpu.VMEM((nch, CHUNK), I32))
        scratch.insert(8, pltpu.VMEM_SHARED((Nb,), F32))

    @functools.partial(
        pl.kernel, mesh=mesh,
        out_type=[jax.ShapeDtypeStruct((NW, nch, CHUNK), F32),
                  jax.ShapeDtypeStruct((NC * Npad,), F32)],
        compiler_params=pltpu.CompilerParams(needs_layout_passes=False, use_tc_tiling_on_sc=False),
        scratch_types=scratch)
    def k(*refs):
        if two:
            (idxa_h, idxb_h, dst_h, ta_h, tb_h, w_h, den_h,
             idxa_v, idxb_v, dst_v, va, vb, wbuf, zbuf,
             tash, tbsh, densh, sem) = refs
        else:
            (idxa_h, dst_h, ta_h, w_h, den_h,
             idxa_v, dst_v, va, vb, wbuf, zbuf,
             tash, densh, sem) = refs
        c = lax.axis_index("c")
        s = lax.axis_index("s")
        g = c * NS + s

        pltpu.sync_copy(ta_h.at[pl.ds(s * ta_str, ta_str)],
                        zbuf.at[pl.ds(0, ta_str)])
        pltpu.sync_copy(zbuf.at[pl.ds(0, ta_str)],
                        tash.at[pl.ds(s * ta_str, ta_str)])
        if two:
            pltpu.sync_copy(tb_h.at[pl.ds(s * tb_str, tb_str)],
                            zbuf.at[pl.ds(0, tb_str)])
            pltpu.sync_copy(zbuf.at[pl.ds(0, tb_str)],
                            tbsh.at[pl.ds(s * tb_str, tb_str)])
        zv = jnp.zeros((16,), F32)

        def zfill(i, _):
            zbuf[pl.ds(i * 16, 16)] = zv
            return 0
        lax.fori_loop(0, stripe // 16, zfill, 0)
        pltpu.sync_copy(zbuf.at[pl.ds(0, stripe)],
                        densh.at[pl.ds(s * stripe, stripe)])
        plsc.subcore_barrier()

        pltpu.sync_copy(idxa_h.at[g], idxa_v)
        if two:
            pltpu.sync_copy(idxb_h.at[g], idxb_v)
        pltpu.sync_copy(dst_h.at[g], dst_v)

        def chunk_body(j, _):
            cpa = pltpu.async_copy(tash.at[idxa_v.at[j]], va, sem)
            cpb = (pltpu.async_copy(tbsh.at[idxb_v.at[j]], vb, sem)
                   if two else None)
            cpa.wait()
            if two:
                cpb.wait()
            for m in range(CHUNK // 16):
                e = va[pl.ds(m * 16, 16)]
                if two:
                    e = e + vb[pl.ds(m * 16, 16)]
                e = jnp.where(e >= 0.0, e, 0.2 * e)
                wbuf[pl.ds(m * 16, 16)] = jnp.exp(e)
            pltpu.sync_copy(wbuf, densh.at[dst_v.at[j]], add=True)
            pltpu.sync_copy(wbuf, w_h.at[g, j])
            return 0
        lax.fori_loop(0, nch, chunk_body, 0)
        plsc.subcore_barrier()
        pltpu.sync_copy(densh.at[pl.ds(s * stripe, stripe)],
                        zbuf.at[pl.ds(0, stripe)])
        pltpu.sync_copy(zbuf.at[pl.ds(0, stripe)],
                        den_h.at[pl.ds(c * Npad + s * stripe, stripe)])

    return k


def _weighted_scatter(nch, Npad, SW, nrng=1):
    mesh = plsc.VectorSubcoreMesh(core_axis_name="c", subcore_axis_name="s")
    nslab = 64 // SW
    rpv = 16 // SW
    half = Npad // nrng
    stripe = half // NS
    zbr = stripe // 8

    @functools.partial(
        pl.kernel, mesh=mesh,
        out_type=jax.ShapeDtypeStruct((NC, nslab, Npad, SW), F32),
        compiler_params=pltpu.CompilerParams(needs_layout_passes=False, use_tc_tiling_on_sc=False),
        scratch_types=[pltpu.VMEM((nch, CHUNK), I32),
                       pltpu.VMEM((nrng * nch, CHUNK), I32),
                       pltpu.VMEM((nch, CHUNK), F32),
                       pltpu.VMEM((CHUNK,), I32),
                       pltpu.VMEM((CHUNK,), I32),
                       pltpu.VMEM((CHUNK, SW), F32),
                       pltpu.VMEM((CHUNK, SW), F32),
                       pltpu.VMEM((zbr, SW), F32),
                       pltpu.VMEM((zbr, SW), F32),
                       pltpu.VMEM_SHARED((half + 8, SW), F32),
                       pltpu.SemaphoreType.DMA,
                       pltpu.SemaphoreType.DMA])
    def k(src_h, dst_h, w_h, h8_h, out_h,
          src_v, dst_v, w_v, idx8a, idx8b, rows_a, rows_b,
          zbuf, dbuf, accsh, sema, semb):
        c = lax.axis_index("c")
        s = lax.axis_index("s")
        g = c * NS + s

        pltpu.sync_copy(src_h.at[g], src_v)
        pltpu.sync_copy(dst_h.at[g], dst_v)
        pltpu.sync_copy(w_h.at[g], w_v)

        lanes = lax.broadcasted_iota(I32, (16,), 0)
        colpat = lanes % SW
        rowpat = lanes // SW
        zv = jnp.zeros((16,), F32)

        def zfill(i, r):
            plsc.store_scatter(zbuf, [r, colpat], zv)
            return r + rpv
        lax.fori_loop(0, (zbr * SW) // 16, zfill, rowpat)

        for p in range(nslab):
            for r in range(nrng):
                def zcp(q, _):
                    pltpu.sync_copy(
                        zbuf, accsh.at[pl.ds(s * stripe + q * zbr, zbr)])
                    return 0
                lax.fori_loop(0, 8, zcp, 0)
                plsc.subcore_barrier()

                def scale_scatter(rows, jv, dj, p=p, r=r):
                    def scale(m, r_idx):
                        for u in range(4):
                            ri = r_idx + u * rpv
                            v = plsc.load_gather(rows, [ri, colpat])
                            wv = plsc.load_gather(w_v, [jv, ri])
                            plsc.store_scatter(rows, [ri, colpat], v * wv)
                        return r_idx + 4 * rpv
                    lax.fori_loop(0, CHUNK // rpv // 4, scale, rowpat)
                    pltpu.sync_copy(rows, accsh.at[dst_v.at[r * nch + dj]],
                                    add=True)

                def pair_body(jj, jv, p=p, r=r):
                    j0 = 2 * jj
                    j1 = j0 + 1
                    for m in range(CHUNK // 16):
                        sl = pl.ds(m * 16, 16)
                        idx8a[sl] = src_v[j0, sl] * nslab + p
                    cpa = pltpu.async_copy(h8_h.at[idx8a], rows_a, sema)
                    for m in range(CHUNK // 16):
                        sl = pl.ds(m * 16, 16)
                        idx8b[sl] = src_v[j1, sl] * nslab + p
                    cpb = pltpu.async_copy(h8_h.at[idx8b], rows_b, semb)
                    cpa.wait()
                    scale_scatter(rows_a, jv, j0)
                    cpb.wait()
                    scale_scatter(rows_b, jv + 1, j1)
                    return jv + 2
                lax.fori_loop(0, nch // 2, pair_body, jnp.zeros((16,), I32))
                if nch % 2:
                    j = nch - 1
                    for m in range(CHUNK // 16):
                        sl = pl.ds(m * 16, 16)
                        idx8a[sl] = src_v[j, sl] * nslab + p
                    pltpu.async_copy(h8_h.at[idx8a], rows_a, sema).wait()
                    scale_scatter(rows_a, jnp.full((16,), j, I32), j)
                plsc.subcore_barrier()

                def dcp(q, _, p=p, r=r):
                    off = s * stripe + q * zbr
                    pltpu.sync_copy(accsh.at[pl.ds(off, zbr)], dbuf)
                    pltpu.sync_copy(
                        dbuf, out_h.at[c, p, pl.ds(r * half + off, zbr)])
                    return 0
                lax.fori_loop(0, 8, dcp, 0)

    return k


def _unpool(nrch, Nsm, NBpad, with_s):
    mesh = plsc.VectorSubcoreMesh(core_axis_name="c", subcore_axis_name="s")

    out_type = [jax.ShapeDtypeStruct((NBpad, 64), F32)]
    scratch = [pltpu.VMEM((nrch, CHUNK), I32),
               pltpu.VMEM((CHUNK, 64), F32),
               pltpu.VMEM((CHUNK, 64), F32),
               pltpu.SemaphoreType.DMA]
    if with_s:
        out_type.append(jax.ShapeDtypeStruct((NBpad, 16), F32))
        scratch.insert(3, pltpu.VMEM((CHUNK, 16), F32))
        scratch.insert(4, pltpu.VMEM((CHUNK, 16), F32))

    @functools.partial(pl.kernel, mesh=mesh, out_type=out_type,
                       compiler_params=pltpu.CompilerParams(
                           needs_layout_passes=False,
                           use_tc_tiling_on_sc=False),
                       scratch_types=scratch)
    def k(*refs):
        if with_s:
            (o2n_h, z_h, za_h, u_h, ub_h, h2_h, s2_h,
             o2n_v, zrows, urows, zs, ubs, sem) = refs
        else:
            (o2n_h, z_h, u_h, h2_h, o2n_v, zrows, urows, sem) = refs
        c = lax.axis_index("c")
        s = lax.axis_index("s")
        g = c * NS + s
        pltpu.sync_copy(o2n_h.at[g], o2n_v)

        lanes = lax.broadcasted_iota(I32, (16,), 0)
        z16 = jnp.zeros((16,), I32)

        def chunk_body(j, _):
            base = (g * nrch + j) * CHUNK
            pltpu.async_copy(z_h.at[o2n_v.at[j]], zrows, sem).wait()
            pltpu.sync_copy(u_h.at[pl.ds(base, CHUNK)], urows)

            def addv(m, carry):
                rowv, colv = carry
                zv = plsc.load_gather(zrows, [rowv, colv])
                uv = plsc.load_gather(urows, [rowv, colv])
                plsc.store_scatter(zrows, [rowv, colv], zv + uv)
                colv2 = colv + 16
                wrap = colv2 >= 64
                return (jnp.where(wrap, rowv + 1, rowv),
                        jnp.where(wrap, colv2 - 64, colv2))
            lax.fori_loop(0, CHUNK * 4, addv, (z16, lanes))
            pltpu.sync_copy(zrows, h2_h.at[pl.ds(base, CHUNK)])
            if with_s:
                pltpu.async_copy(za_h.at[o2n_v.at[j]], zs, sem).wait()
                pltpu.sync_copy(ub_h.at[pl.ds(base, CHUNK)], ubs)

                def addv2(m, rowv):
                    zv = plsc.load_gather(zs, [rowv, lanes])
                    uv = plsc.load_gather(ubs, [rowv, lanes])
                    plsc.store_scatter(zs, [rowv, lanes], zv + uv)
                    return rowv + 1
                lax.fori_loop(0, CHUNK, addv2, z16)
                pltpu.sync_copy(zs, s2_h.at[pl.ds(base, CHUNK)])
            return 0
        lax.fori_loop(0, nrch, chunk_body, 0)

    return k


def _pad_rows(x, npad):
    return jnp.pad(x, ((0, npad - x.shape[0]), (0, 0)))


def _pad_edges(idx, epad, fill):
    e = idx.shape[0]
    out = jnp.pad(idx, (0, epad - e), constant_values=fill)
    return out.reshape(NW, epad // (NW * CHUNK), CHUNK)


def _avec(a1, a2):
    return jnp.pad(jnp.stack([a1, a2], axis=1), ((0, 0), (0, 14)))


def _range_dst(dst_r, nrng, Npad):
    if nrng == 1:
        return dst_r
    half = Npad // nrng
    junk = half + (lax.broadcasted_iota(I32, dst_r.shape, 2) % 8)
    parts = []
    for r in range(nrng):
        dv = dst_r - r * half
        ok = jnp.logical_and(dv >= 0, dv < half)
        parts.append(jnp.where(ok, dv, junk))
    out = jnp.stack(parts, axis=1)
    return out.reshape(dst_r.shape[0], -1, dst_r.shape[2])


def _gat_conv(src_r, dst_r, idxa_r, h, s1, s2, Npad, N, tvec, SW, nrng=1):
    nch = src_r.shape[1]
    nslab = 64 // SW
    two = s2 is not None
    if two:
        ew = _edge_weights(nch, s1.shape[0], s2.shape[0], Npad, True)
        w, den = ew(idxa_r, dst_r, dst_r, s1, s2)
    else:
        ew = _edge_weights(nch, s1.shape[0], 0, Npad, False)
        w, den = ew(idxa_r, dst_r, s1)
    h8 = h.reshape(Npad * nslab, SW)
    num = _weighted_scatter(nch, Npad, SW, nrng)(
        src_r, _range_dst(dst_r, nrng, Npad), w, h8)
    return _finish(num, den, tvec, N)


def kernel(x_primal, x_dual, edge_index_primal, edge_index_dual,
           primal_edge_to_dual_node_idx, old_to_new_primal, old_to_new_dual,
           x_primal_before, x_dual_before, edge_index_primal_before,
           edge_index_dual_before, primal_edge_to_dual_node_idx_before,
           W_p1, W_d1, a_d1, a_p1, W_p2, W_d2, a_d2, a_p2,
           g_p1, b_p1, g_d1, b_d1, g_p2, b_p2, g_d2, b_d2):
    N_P, C = x_primal.shape
    N_D = x_dual.shape[0]
    N_PB = x_primal_before.shape[0]
    N_DB = x_dual_before.shape[0]
    EB = NW * CHUNK
    NPp = _align(N_P + 1, 1024)
    NDp = _align(N_D + 1, 1024)
    NPBp = _align(N_PB + 1, EB)
    NDBp = _align(N_DB + 1, EB)

    src_d = _pad_edges(edge_index_dual[0], _align(edge_index_dual.shape[1], EB), 0)
    dst_d = _pad_edges(edge_index_dual[1], _align(edge_index_dual.shape[1], EB), N_D)
    src_p = _pad_edges(edge_index_primal[0], _align(edge_index_primal.shape[1], EB), 0)
    dst_p = _pad_edges(edge_index_primal[1], _align(edge_index_primal.shape[1], EB), N_P)
    p2d = _pad_edges(primal_edge_to_dual_node_idx, _align(primal_edge_to_dual_node_idx.shape[0], EB), 0)
    src_db = _pad_edges(edge_index_dual_before[0], _align(edge_index_dual_before.shape[1], EB), 0)
    dst_db = _pad_edges(edge_index_dual_before[1], _align(edge_index_dual_before.shape[1], EB), N_DB)
    src_pb = _pad_edges(edge_index_primal_before[0], _align(edge_index_primal_before.shape[1], EB), 0)
    dst_pb = _pad_edges(edge_index_primal_before[1], _align(edge_index_primal_before.shape[1], EB), N_PB)
    p2db = _pad_edges(primal_edge_to_dual_node_idx_before, _align(primal_edge_to_dual_node_idx_before.shape[0], EB), 0)
    o2n_p = _pad_edges(old_to_new_primal, NPBp, 0)
    o2n_d = _pad_edges(old_to_new_dual, NDBp, 0)

    A0 = jnp.zeros((64, 16), F32)

    hd, Sd = _embed(_pad_rows(x_dual, NDp), W_d1, _avec(a_d1[:64], a_d1[64:]))
    yd, td, std = _gat_conv(src_d, dst_d, src_d, hd, Sd[:, 0], Sd[:, 1],
                            NDp, N_D, a_p1, 8)
    hp, _ = _embed(_pad_rows(x_primal, NPp), W_p1, A0)
    yp, _, stp = _gat_conv(src_p, dst_p, p2d, hp, td.reshape(NDp), None,
                           NPp, N_P, a_p1, 16)

    zd, zad = _bn_relu(yd, std, g_d1, b_d1, N_D,
                       W2=W_d2[:64], A2=_avec(a_d2[:64], a_d2[64:]))
    zp, _ = _bn_relu(yp, stp, g_p1, b_p1, N_P, W2=W_p2[:64], A2=A0)
    ud, Ubd = _embed(_pad_rows(x_dual_before, NDBp), W_d2[64:],
                     _avec(a_d2[:64], a_d2[64:]))
    up, _ = _embed(_pad_rows(x_primal_before, NPBp), W_p2[64:], A0)

    hd2, Sd2 = _unpool(NDBp // EB, NDp, NDBp, True)(o2n_d, zd, zad, ud, Ubd)
    (hp2,) = _unpool(NPBp // EB, NPp, NPBp, False)(o2n_p, zp, up)

    yd2, t2, std2 = _gat_conv(src_db, dst_db, src_db, hd2,
                              Sd2[:, 0], Sd2[:, 1], NDBp, N_DB, a_p2, 8,
                              nrng=2)
    yp2, _, stp2 = _gat_conv(src_pb, dst_pb, p2db, hp2, t2.reshape(NDBp),
                             None, NPBp, N_PB, a_p2, 16)

    d2 = _bn_relu(yd2, std2, g_d2, b_d2, N_DB)[:N_DB]
    p2 = _bn_relu(yp2, stp2, g_p2, b_p2, N_PB)[:N_PB]
    return p2, d2


# speedup vs baseline: 1.0156x; 1.0156x over previous
import functools

import jax
import jax.numpy as jnp
from jax import lax
from jax.experimental import pallas as pl
from jax.experimental.pallas import tpu as pltpu
from jax.experimental.pallas import tpu_sc as plsc

F32 = jnp.float32
I32 = jnp.int32
NC, NS, NLANE = 2, 16, 16
NW = NC * NS
CHUNK = 128


def _align(n, m):
    return ((n + m - 1) // m) * m


def _embed(x, W, A, blk=1024):
    Np, K = x.shape

    def body(x_ref, w_ref, a_ref, h_ref, s_ref):
        h = jnp.dot(x_ref[...], w_ref[...], preferred_element_type=F32)
        h_ref[...] = h
        s_ref[...] = jnp.dot(h, a_ref[...], preferred_element_type=F32)

    return pl.pallas_call(
        body,
        grid=(Np // blk,),
        in_specs=[pl.BlockSpec((blk, K), lambda i: (i, 0)),
                  pl.BlockSpec((K, 64), lambda i: (0, 0)),
                  pl.BlockSpec((64, 16), lambda i: (0, 0))],
        out_specs=[pl.BlockSpec((blk, 64), lambda i: (i, 0)),
                   pl.BlockSpec((blk, 16), lambda i: (i, 0))],
        out_shape=[jax.ShapeDtypeStruct((Np, 64), F32),
                   jax.ShapeDtypeStruct((Np, 16), F32)],
    )(x, W, A)


def _finish(numer4, denom, tvec, n_real, blk=1024):
    _, nslab, Np, SW = numer4.shape
    grid = Np // blk

    def body(*refs):
        nrefs = refs[:nslab]
        d_ref, tv_ref, y_ref, t_ref, st_ref = refs[nslab:]
        i = pl.program_id(0)
        n = jnp.concatenate([r[0, 0] + r[1, 0] for r in nrefs], axis=-1)
        d = d_ref[0] + d_ref[1] + 1e-16
        y = n / d
        y_ref[...] = y
        t_ref[...] = jnp.dot(y, tv_ref[...], preferred_element_type=F32)
        rows = i * blk + lax.broadcasted_iota(I32, (blk, 64), 0)
        ym = jnp.where(rows < n_real, y, 0.0)

        @pl.when(i == 0)
        def _():
            st_ref[...] = jnp.zeros((8, 128), F32)

        st_ref[0:1, 0:64] += jnp.sum(ym, axis=0, keepdims=True)
        st_ref[1:2, 0:64] += jnp.sum(ym * ym, axis=0, keepdims=True)

    in_specs = [pl.BlockSpec((NC, 1, blk, SW), (lambda i, p=p: (0, p, i, 0)))
                for p in range(nslab)]
    in_specs += [pl.BlockSpec((2, blk, 1), lambda i: (0, i, 0)),
                 pl.BlockSpec((64, 1), lambda i: (0, 0))]
    return pl.pallas_call(
        body,
        grid=(grid,),
        in_specs=in_specs,
        out_specs=[pl.BlockSpec((blk, 64), lambda i: (i, 0)),
                   pl.BlockSpec((blk, 1), lambda i: (i, 0)),
                   pl.BlockSpec((8, 128), lambda i: (0, 0))],
        out_shape=[jax.ShapeDtypeStruct((Np, 64), F32),
                   jax.ShapeDtypeStruct((Np, 1), F32),
                   jax.ShapeDtypeStruct((8, 128), F32)],
    )(*([numer4] * nslab), denom.reshape(2, Np, 1), tvec.reshape(64, 1))


def _bn_relu(y, stats, g, b, n_real, W2=None, A2=None, blk=1024):
    Np, _ = y.shape
    project = W2 is not None
    inv_n = 1.0 / float(n_real)

    def body(y_ref, st_ref, g_ref, b_ref, *rest):
        if project:
            w_ref, a_ref, z_ref, za_ref = rest
        else:
            (z_ref,) = rest
        s = st_ref[0:1, 0:64]
        sq = st_ref[1:2, 0:64]
        m = s * inv_n
        var = sq * inv_n - m * m
        r = lax.rsqrt(var + 1e-5)
        p = jnp.maximum((y_ref[...] - m) * (r * g_ref[...]) + b_ref[...], 0.0)
        if project:
            z = jnp.dot(p, w_ref[...], preferred_element_type=F32)
            z_ref[...] = z
            za_ref[...] = jnp.dot(z, a_ref[...], preferred_element_type=F32)
        else:
            z_ref[...] = p

    in_specs = [pl.BlockSpec((blk, 64), lambda i: (i, 0)),
                pl.BlockSpec((8, 128), lambda i: (0, 0)),
                pl.BlockSpec((1, 64), lambda i: (0, 0)),
                pl.BlockSpec((1, 64), lambda i: (0, 0))]
    args = [y, stats, g.reshape(1, 64), b.reshape(1, 64)]
    out_specs = [pl.BlockSpec((blk, 64), lambda i: (i, 0))]
    out_shape = [jax.ShapeDtypeStruct((Np, 64), F32)]
    if project:
        in_specs += [pl.BlockSpec((64, 64), lambda i: (0, 0)),
                     pl.BlockSpec((64, 16), lambda i: (0, 0))]
        args += [W2, A2]
        out_specs += [pl.BlockSpec((blk, 16), lambda i: (i, 0))]
        out_shape += [jax.ShapeDtypeStruct((Np, 16), F32)]

    res = pl.pallas_call(
        body, grid=(Np // blk,), in_specs=in_specs, out_specs=out_specs,
        out_shape=out_shape)(*args)
    return res if project else res[0]


def _edge_weights(nch, Na, Nb, Npad, two):
    mesh = plsc.VectorSubcoreMesh(core_axis_name="c", subcore_axis_name="s")
    stripe = Npad // NS
    ta_str = Na // NS
    tb_str = (Nb // NS) if two else 0

    tsz = max(stripe, ta_str, tb_str)
    scratch = [pltpu.VMEM((nch, CHUNK), I32),
               pltpu.VMEM((nch, CHUNK), I32),
               pltpu.VMEM((CHUNK,), F32),
               pltpu.VMEM((CHUNK,), F32),
               pltpu.VMEM((CHUNK,), F32),
               pltpu.VMEM((tsz,), F32),
               pltpu.VMEM_SHARED((Na,), F32),
               pltpu.VMEM_SHARED((Npad,), F32),
               pltpu.SemaphoreType.DMA]
    if two:
        scratch.insert(1, pltpu.VMEM((nch, CHUNK), I32))
        scratch.insert(8, pltpu.VMEM_SHARED((Nb,), F32))

    @functools.partial(
        pl.kernel, mesh=mesh,
        out_type=[jax.ShapeDtypeStruct((NW, nch, CHUNK), F32),
                  jax.ShapeDtypeStruct((NC * Npad,), F32)],
        compiler_params=pltpu.CompilerParams(needs_layout_passes=False, use_tc_tiling_on_sc=False),
        scratch_types=scratch)
    def k(*refs):
        if two:
            (idxa_h, idxb_h, dst_h, ta_h, tb_h, w_h, den_h,
             idxa_v, idxb_v, dst_v, va, vb, wbuf, zbuf,
             tash, tbsh, densh, sem) = refs
        else:
            (idxa_h, dst_h, ta_h, w_h, den_h,
             idxa_v, dst_v, va, vb, wbuf, zbuf,
             tash, densh, sem) = refs
        c = lax.axis_index("c")
        s = lax.axis_index("s")
        g = c * NS + s

        pltpu.sync_copy(ta_h.at[pl.ds(s * ta_str, ta_str)],
                        zbuf.at[pl.ds(0, ta_str)])
        pltpu.sync_copy(zbuf.at[pl.ds(0, ta_str)],
                        tash.at[pl.ds(s * ta_str, ta_str)])
        if two:
            pltpu.sync_copy(tb_h.at[pl.ds(s * tb_str, tb_str)],
                            zbuf.at[pl.ds(0, tb_str)])
            pltpu.sync_copy(zbuf.at[pl.ds(0, tb_str)],
                            tbsh.at[pl.ds(s * tb_str, tb_str)])
        zv = jnp.zeros((16,), F32)

        def zfill(i, _):
            zbuf[pl.ds(i * 16, 16)] = zv
            return 0
        lax.fori_loop(0, stripe // 16, zfill, 0)
        pltpu.sync_copy(zbuf.at[pl.ds(0, stripe)],
                        densh.at[pl.ds(s * stripe, stripe)])
        plsc.subcore_barrier()

        pltpu.sync_copy(idxa_h.at[g], idxa_v)
        if two:
            pltpu.sync_copy(idxb_h.at[g], idxb_v)
        pltpu.sync_copy(dst_h.at[g], dst_v)

        def chunk_body(j, _):
            cpa = pltpu.async_copy(tash.at[idxa_v.at[j]], va, sem)
            cpb = (pltpu.async_copy(tbsh.at[idxb_v.at[j]], vb, sem)
                   if two else None)
            cpa.wait()
            if two:
                cpb.wait()
            for m in range(CHUNK // 16):
                e = va[pl.ds(m * 16, 16)]
                if two:
                    e = e + vb[pl.ds(m * 16, 16)]
                e = jnp.where(e >= 0.0, e, 0.2 * e)
                wbuf[pl.ds(m * 16, 16)] = jnp.exp(e)
            pltpu.sync_copy(wbuf, densh.at[dst_v.at[j]], add=True)
            pltpu.sync_copy(wbuf, w_h.at[g, j])
            return 0
        lax.fori_loop(0, nch, chunk_body, 0)
        plsc.subcore_barrier()
        pltpu.sync_copy(densh.at[pl.ds(s * stripe, stripe)],
                        zbuf.at[pl.ds(0, stripe)])
        pltpu.sync_copy(zbuf.at[pl.ds(0, stripe)],
                        den_h.at[pl.ds(c * Npad + s * stripe, stripe)])

    return k


def _weighted_scatter(nch, Npad, SW, nrng=1):
    mesh = plsc.VectorSubcoreMesh(core_axis_name="c", subcore_axis_name="s")
    nslab = 64 // SW
    rpv = 16 // SW
    half = Npad // nrng
    stripe = half // NS
    zbr = stripe // 8

    @functools.partial(
        pl.kernel, mesh=mesh,
        out_type=jax.ShapeDtypeStruct((NC, nslab, Npad, SW), F32),
        compiler_params=pltpu.CompilerParams(needs_layout_passes=False, use_tc_tiling_on_sc=False),
        scratch_types=[pltpu.VMEM((nch, CHUNK), I32),
                       pltpu.VMEM((nrng * nch, CHUNK), I32),
                       pltpu.VMEM((nch, CHUNK), F32),
                       pltpu.VMEM((CHUNK,), I32),
                       pltpu.VMEM((CHUNK,), I32),
                       pltpu.VMEM((CHUNK, SW), F32),
                       pltpu.VMEM((CHUNK, SW), F32),
                       pltpu.VMEM((zbr, SW), F32),
                       pltpu.VMEM((zbr, SW), F32),
                       pltpu.VMEM_SHARED((half + 8, SW), F32),
                       pltpu.SemaphoreType.DMA,
                       pltpu.SemaphoreType.DMA,
                       pltpu.SemaphoreType.DMA,
                       pltpu.SemaphoreType.DMA])
    def k(src_h, dst_h, w_h, h8_h, out_h,
          src_v, dst_v, w_v, idx8a, idx8b, rows_a, rows_b,
          zbuf, dbuf, accsh, sema, semb, semc, semd):
        c = lax.axis_index("c")
        s = lax.axis_index("s")
        g = c * NS + s

        pltpu.sync_copy(src_h.at[g], src_v)
        pltpu.sync_copy(dst_h.at[g], dst_v)
        pltpu.sync_copy(w_h.at[g], w_v)

        lanes = lax.broadcasted_iota(I32, (16,), 0)
        colpat = lanes % SW
        rowpat = lanes // SW
        zv = jnp.zeros((16,), F32)

        def zfill(i, r):
            plsc.store_scatter(zbuf, [r, colpat], zv)
            return r + rpv
        lax.fori_loop(0, (zbr * SW) // 16, zfill, rowpat)

        for p in range(nslab):
            for r in range(nrng):
                def zcp(q, _):
                    pltpu.sync_copy(
                        zbuf, accsh.at[pl.ds(s * stripe + q * zbr, zbr)])
                    return 0
                lax.fori_loop(0, 8, zcp, 0)
                plsc.subcore_barrier()

                def scale(rows, jv):
                    def body(m, r_idx):
                        for u in range(4):
                            ri = r_idx + u * rpv
                            v = plsc.load_gather(rows, [ri, colpat])
                            wv = plsc.load_gather(w_v, [jv, ri])
                            plsc.store_scatter(rows, [ri, colpat], v * wv)
                        return r_idx + 4 * rpv
                    lax.fori_loop(0, CHUNK // rpv // 4, body, rowpat)

                def pair_body(jj, jv, p=p, r=r):
                    j0 = 2 * jj
                    j1 = j0 + 1
                    for m in range(CHUNK // 16):
                        sl = pl.ds(m * 16, 16)
                        idx8a[sl] = src_v[j0, sl] * nslab + p
                    cpa = pltpu.async_copy(h8_h.at[idx8a], rows_a, sema)
                    for m in range(CHUNK // 16):
                        sl = pl.ds(m * 16, 16)
                        idx8b[sl] = src_v[j1, sl] * nslab + p
                    cpb = pltpu.async_copy(h8_h.at[idx8b], rows_b, semb)
                    cpa.wait()
                    scale(rows_a, jv)
                    sca = pltpu.async_copy(
                        rows_a, accsh.at[dst_v.at[r * nch + j0]], semc,
                        add=True)
                    cpb.wait()
                    scale(rows_b, jv + 1)
                    scb = pltpu.async_copy(
                        rows_b, accsh.at[dst_v.at[r * nch + j1]], semd,
                        add=True)
                    sca.wait()
                    scb.wait()
                    return jv + 2
                lax.fori_loop(0, nch // 2, pair_body, jnp.zeros((16,), I32))
                if nch % 2:
                    j = nch - 1
                    for m in range(CHUNK // 16):
                        sl = pl.ds(m * 16, 16)
                        idx8a[sl] = src_v[j, sl] * nslab + p
                    pltpu.async_copy(h8_h.at[idx8a], rows_a, sema).wait()
                    scale(rows_a, jnp.full((16,), j, I32))
                    pltpu.sync_copy(rows_a, accsh.at[dst_v.at[r * nch + j]],
                                    add=True)
                plsc.subcore_barrier()

                def dcp(q, _, p=p, r=r):
                    off = s * stripe + q * zbr
                    pltpu.sync_copy(accsh.at[pl.ds(off, zbr)], dbuf)
                    pltpu.sync_copy(
                        dbuf, out_h.at[c, p, pl.ds(r * half + off, zbr)])
                    return 0
                lax.fori_loop(0, 8, dcp, 0)

    return k


def _unpool(nrch, Nsm, NBpad, with_s):
    mesh = plsc.VectorSubcoreMesh(core_axis_name="c", subcore_axis_name="s")

    out_type = [jax.ShapeDtypeStruct((NBpad, 64), F32)]
    scratch = [pltpu.VMEM((nrch, CHUNK), I32),
               pltpu.VMEM((CHUNK, 64), F32),
               pltpu.VMEM((CHUNK, 64), F32),
               pltpu.SemaphoreType.DMA]
    if with_s:
        out_type.append(jax.ShapeDtypeStruct((NBpad, 16), F32))
        scratch.insert(3, pltpu.VMEM((CHUNK, 16), F32))
        scratch.insert(4, pltpu.VMEM((CHUNK, 16), F32))

    @functools.partial(pl.kernel, mesh=mesh, out_type=out_type,
                       compiler_params=pltpu.CompilerParams(
                           needs_layout_passes=False,
                           use_tc_tiling_on_sc=False),
                       scratch_types=scratch)
    def k(*refs):
        if with_s:
            (o2n_h, z_h, za_h, u_h, ub_h, h2_h, s2_h,
             o2n_v, zrows, urows, zs, ubs, sem) = refs
        else:
            (o2n_h, z_h, u_h, h2_h, o2n_v, zrows, urows, sem) = refs
        c = lax.axis_index("c")
        s = lax.axis_index("s")
        g = c * NS + s
        pltpu.sync_copy(o2n_h.at[g], o2n_v)

        lanes = lax.broadcasted_iota(I32, (16,), 0)
        z16 = jnp.zeros((16,), I32)

        def chunk_body(j, _):
            base = (g * nrch + j) * CHUNK
            pltpu.async_copy(z_h.at[o2n_v.at[j]], zrows, sem).wait()
            pltpu.sync_copy(u_h.at[pl.ds(base, CHUNK)], urows)

            def addv(m, carry):
                rowv, colv = carry
                zv = plsc.load_gather(zrows, [rowv, colv])
                uv = plsc.load_gather(urows, [rowv, colv])
                plsc.store_scatter(zrows, [rowv, colv], zv + uv)
                colv2 = colv + 16
                wrap = colv2 >= 64
                return (jnp.where(wrap, rowv + 1, rowv),
                        jnp.where(wrap, colv2 - 64, colv2))
            lax.fori_loop(0, CHUNK * 4, addv, (z16, lanes))
            pltpu.sync_copy(zrows, h2_h.at[pl.ds(base, CHUNK)])
            if with_s:
                pltpu.async_copy(za_h.at[o2n_v.at[j]], zs, sem).wait()
                pltpu.sync_copy(ub_h.at[pl.ds(base, CHUNK)], ubs)

                def addv2(m, rowv):
                    zv = plsc.load_gather(zs, [rowv, lanes])
                    uv = plsc.load_gather(ubs, [rowv, lanes])
                    plsc.store_scatter(zs, [rowv, lanes], zv + uv)
                    return rowv + 1
                lax.fori_loop(0, CHUNK, addv2, z16)
                pltpu.sync_copy(zs, s2_h.at[pl.ds(base, CHUNK)])
            return 0
        lax.fori_loop(0, nrch, chunk_body, 0)

    return k


def _pad_rows(x, npad):
    return jnp.pad(x, ((0, npad - x.shape[0]), (0, 0)))


def _pad_edges(idx, epad, fill):
    e = idx.shape[0]
    out = jnp.pad(idx, (0, epad - e), constant_values=fill)
    return out.reshape(NW, epad // (NW * CHUNK), CHUNK)


def _avec(a1, a2):
    return jnp.pad(jnp.stack([a1, a2], axis=1), ((0, 0), (0, 14)))


def _range_dst(dst_r, nrng, Npad):
    if nrng == 1:
        return dst_r
    half = Npad // nrng
    junk = half + (lax.broadcasted_iota(I32, dst_r.shape, 2) % 8)
    parts = []
    for r in range(nrng):
        dv = dst_r - r * half
        ok = jnp.logical_and(dv >= 0, dv < half)
        parts.append(jnp.where(ok, dv, junk))
    out = jnp.stack(parts, axis=1)
    return out.reshape(dst_r.shape[0], -1, dst_r.shape[2])


def _gat_conv(src_r, dst_r, idxa_r, h, s1, s2, Npad, N, tvec, SW, nrng=1):
    nch = src_r.shape[1]
    nslab = 64 // SW
    two = s2 is not None
    if two:
        ew = _edge_weights(nch, s1.shape[0], s2.shape[0], Npad, True)
        w, den = ew(idxa_r, dst_r, dst_r, s1, s2)
    else:
        ew = _edge_weights(nch, s1.shape[0], 0, Npad, False)
        w, den = ew(idxa_r, dst_r, s1)
    h8 = h.reshape(Npad * nslab, SW)
    num = _weighted_scatter(nch, Npad, SW, nrng)(
        src_r, _range_dst(dst_r, nrng, Npad), w, h8)
    return _finish(num, den, tvec, N)


def kernel(x_primal, x_dual, edge_index_primal, edge_index_dual,
           primal_edge_to_dual_node_idx, old_to_new_primal, old_to_new_dual,
           x_primal_before, x_dual_before, edge_index_primal_before,
           edge_index_dual_before, primal_edge_to_dual_node_idx_before,
           W_p1, W_d1, a_d1, a_p1, W_p2, W_d2, a_d2, a_p2,
           g_p1, b_p1, g_d1, b_d1, g_p2, b_p2, g_d2, b_d2):
    N_P, C = x_primal.shape
    N_D = x_dual.shape[0]
    N_PB = x_primal_before.shape[0]
    N_DB = x_dual_before.shape[0]
    EB = NW * CHUNK
    NPp = _align(N_P + 1, 1024)
    NDp = _align(N_D + 1, 1024)
    NPBp = _align(N_PB + 1, EB)
    NDBp = _align(N_DB + 1, EB)

    src_d = _pad_edges(edge_index_dual[0], _align(edge_index_dual.shape[1], EB), 0)
    dst_d = _pad_edges(edge_index_dual[1], _align(edge_index_dual.shape[1], EB), N_D)
    src_p = _pad_edges(edge_index_primal[0], _align(edge_index_primal.shape[1], EB), 0)
    dst_p = _pad_edges(edge_index_primal[1], _align(edge_index_primal.shape[1], EB), N_P)
    p2d = _pad_edges(primal_edge_to_dual_node_idx, _align(primal_edge_to_dual_node_idx.shape[0], EB), 0)
    src_db = _pad_edges(edge_index_dual_before[0], _align(edge_index_dual_before.shape[1], EB), 0)
    dst_db = _pad_edges(edge_index_dual_before[1], _align(edge_index_dual_before.shape[1], EB), N_DB)
    src_pb = _pad_edges(edge_index_primal_before[0], _align(edge_index_primal_before.shape[1], EB), 0)
    dst_pb = _pad_edges(edge_index_primal_before[1], _align(edge_index_primal_before.shape[1], EB), N_PB)
    p2db = _pad_edges(primal_edge_to_dual_node_idx_before, _align(primal_edge_to_dual_node_idx_before.shape[0], EB), 0)
    o2n_p = _pad_edges(old_to_new_primal, NPBp, 0)
    o2n_d = _pad_edges(old_to_new_dual, NDBp, 0)

    A0 = jnp.zeros((64, 16), F32)

    hd, Sd = _embed(_pad_rows(x_dual, NDp), W_d1, _avec(a_d1[:64], a_d1[64:]))
    yd, td, std = _gat_conv(src_d, dst_d, src_d, hd, Sd[:, 0], Sd[:, 1],
                            NDp, N_D, a_p1, 8)
    hp, _ = _embed(_pad_rows(x_primal, NPp), W_p1, A0)
    yp, _, stp = _gat_conv(src_p, dst_p, p2d, hp, td.reshape(NDp), None,
                           NPp, N_P, a_p1, 16)

    zd, zad = _bn_relu(yd, std, g_d1, b_d1, N_D,
                       W2=W_d2[:64], A2=_avec(a_d2[:64], a_d2[64:]))
    zp, _ = _bn_relu(yp, stp, g_p1, b_p1, N_P, W2=W_p2[:64], A2=A0)
    ud, Ubd = _embed(_pad_rows(x_dual_before, NDBp), W_d2[64:],
                     _avec(a_d2[:64], a_d2[64:]))
    up, _ = _embed(_pad_rows(x_primal_before, NPBp), W_p2[64:], A0)

    hd2, Sd2 = _unpool(NDBp // EB, NDp, NDBp, True)(o2n_d, zd, zad, ud, Ubd)
    (hp2,) = _unpool(NPBp // EB, NPp, NPBp, False)(o2n_p, zp, up)

    yd2, t2, std2 = _gat_conv(src_db, dst_db, src_db, hd2,
                              Sd2[:, 0], Sd2[:, 1], NDBp, N_DB, a_p2, 8,
                              nrng=2)
    yp2, _, stp2 = _gat_conv(src_pb, dst_pb, p2db, hp2, t2.reshape(NDBp),
                             None, NPBp, N_PB, a_p2, 16)

    d2 = _bn_relu(yd2, std2, g_d2, b_d2, N_DB)[:N_DB]
    p2 = _bn_relu(yp2, stp2, g_p2, b_p2, N_PB)[:N_PB]
    return p2, d2
